# Initial kernel scaffold; baseline (speedup 1.0000x reference)
import jax
import jax.numpy as jnp
from jax.experimental import pallas as pl


def kernel(msg, msg_prev, edge_idx):
    raise NotImplementedError("write your pallas kernel here")


# speedup vs baseline: 2.7646x; 2.7646x over previous
import jax
import jax.numpy as jnp
from jax import lax
from jax.experimental import pallas as pl
from jax.experimental.pallas import tpu as pltpu
from jax.experimental.pallas import tpu_sc as plsc

N_NODES = 10000
N_EDGES = 320000
D = 128
NC = 2
NS = 16
NW = NC * NS
EPW = N_EDGES // NW
CHUNK = 80
NCHUNK = EPW // CHUNK


def _sc_body(msg_hbm, prev_hbm, idx_hbm, out_hbm,
             idx_v, rows_v, prev_v, gsem, psem):
    wid = lax.axis_index("s") * NC + lax.axis_index("c")
    base = wid * EPW

    def step(g, _):
        cb = base + g * CHUNK
        pltpu.sync_copy(idx_hbm.at[pl.ds(cb, CHUNK)], idx_v)
        gcp = pltpu.async_copy(msg_hbm.at[idx_v], rows_v, gsem)
        pcp = pltpu.async_copy(prev_hbm.at[pl.ds(cb, CHUNK)], prev_v, psem)
        pcp.wait()
        pltpu.sync_copy(prev_v, out_hbm.at[pl.ds(cb, CHUNK), pl.ds(0, D)])
        gcp.wait()
        pltpu.sync_copy(rows_v, out_hbm.at[pl.ds(cb, CHUNK), pl.ds(D, D)])
        return 0

    lax.fori_loop(0, NCHUNK, step, 0)


def kernel(msg, msg_prev, edge_idx):
    idx = edge_idx[1].astype(jnp.int32)
    mesh = plsc.VectorSubcoreMesh(
        core_axis_name="c", subcore_axis_name="s",
        num_cores=NC, num_subcores=NS)
    f = pl.kernel(
        _sc_body,
        out_type=jax.ShapeDtypeStruct((N_EDGES, 2 * D), jnp.float32),
        mesh=mesh,
        scratch_types=[
            pltpu.VMEM((CHUNK,), jnp.int32),
            pltpu.VMEM((CHUNK, D), jnp.float32),
            pltpu.VMEM((CHUNK, D), jnp.float32),
            pltpu.SemaphoreType.DMA,
            pltpu.SemaphoreType.DMA,
        ],
    )
    return f(msg, msg_prev, idx)


# speedup vs baseline: 3.5540x; 1.2855x over previous
import jax
import jax.numpy as jnp
from jax import lax
from jax.experimental import pallas as pl
from jax.experimental.pallas import tpu as pltpu
from jax.experimental.pallas import tpu_sc as plsc

N_NODES = 10000
N_EDGES = 320000
D = 128
NC = 2
NS = 16
NW = NC * NS
EPW = N_EDGES // NW
CHUNK = 128
NFULL = EPW // CHUNK
TAIL = EPW - NFULL * CHUNK


def _sc_body(msg_hbm, prev_hbm, idx_hbm, out_hbm,
             idx_v, comb, tail_comb, gsem, psem):
    wid = lax.axis_index("s") * NC + lax.axis_index("c")
    base = wid * EPW
    pltpu.sync_copy(idx_hbm.at[pl.ds(base, EPW)], idx_v)

    def step(g, _):
        cb = base + g * CHUNK
        pcp = pltpu.async_copy(
            prev_hbm.at[pl.ds(cb, CHUNK)], comb.at[:, pl.ds(0, D)], psem)
        gcp = pltpu.async_copy(
            msg_hbm.at[idx_v.at[pl.ds(g * CHUNK, CHUNK)]],
            comb.at[:, pl.ds(D, D)], gsem)
        pcp.wait()
        gcp.wait()
        pltpu.sync_copy(comb, out_hbm.at[pl.ds(cb, CHUNK)])
        return 0

    lax.fori_loop(0, NFULL, step, 0)

    tb = base + NFULL * CHUNK
    pcp = pltpu.async_copy(
        prev_hbm.at[pl.ds(tb, TAIL)], tail_comb.at[:, pl.ds(0, D)], psem)
    gcp = pltpu.async_copy(
        msg_hbm.at[idx_v.at[pl.ds(NFULL * CHUNK, TAIL)]],
        tail_comb.at[:, pl.ds(D, D)], gsem)
    pcp.wait()
    gcp.wait()
    pltpu.sync_copy(tail_comb, out_hbm.at[pl.ds(tb, TAIL)])


def kernel(msg, msg_prev, edge_idx):
    idx = edge_idx[1].astype(jnp.int32)
    mesh = plsc.VectorSubcoreMesh(
        core_axis_name="c", subcore_axis_name="s",
        num_cores=NC, num_subcores=NS)
    f = pl.kernel(
        _sc_body,
        out_type=jax.ShapeDtypeStruct((N_EDGES, 2 * D), jnp.float32),
        mesh=mesh,
        scratch_types=[
            pltpu.VMEM((EPW,), jnp.int32),
            pltpu.VMEM((CHUNK, 2 * D), jnp.float32),
            pltpu.VMEM((TAIL, 2 * D), jnp.float32),
            pltpu.SemaphoreType.DMA,
            pltpu.SemaphoreType.DMA,
        ],
    )
    return f(msg, msg_prev, idx)


# speedup vs baseline: 4.0855x; 1.1496x over previous
import jax
import jax.numpy as jnp
from jax import lax
from jax.experimental import pallas as pl
from jax.experimental.pallas import tpu as pltpu
from jax.experimental.pallas import tpu_sc as plsc

N_NODES = 10000
N_EDGES = 320000
D = 128
NC = 2
NS = 16
NW = NC * NS
EPW = N_EDGES // NW
CHUNK = 128
NFULL = EPW // CHUNK
TAIL = EPW - NFULL * CHUNK


def _sc_body(msg_hbm, prev_hbm, idx_hbm, out_hbm,
             idx_v, comb0, comb1, tail_comb,
             ps0, gs0, ws0, ps1, gs1, ws1):
    wid = lax.axis_index("s") * NC + lax.axis_index("c")
    base = wid * EPW
    pltpu.sync_copy(idx_hbm.at[pl.ds(base, EPW)], idx_v)

    bufs = (comb0, comb1)
    sems = ((ps0, gs0, ws0), (ps1, gs1, ws1))

    def fill_copies(g, k):
        cb = base + g * CHUNK
        buf = bufs[k]
        ps, gs, _ = sems[k]
        return (
            pltpu.make_async_copy(
                prev_hbm.at[pl.ds(cb, CHUNK)], buf.at[:, pl.ds(0, D)], ps),
            pltpu.make_async_copy(
                msg_hbm.at[idx_v.at[pl.ds(g * CHUNK, CHUNK)]],
                buf.at[:, pl.ds(D, D)], gs),
        )

    def scatter_copy(g, k):
        cb = base + g * CHUNK
        return pltpu.make_async_copy(
            bufs[k], out_hbm.at[pl.ds(cb, CHUNK)], sems[k][2])

    def start_fill(g, k):
        for c in fill_copies(g, k):
            c.start()

    def wait_fill(g, k):
        for c in fill_copies(g, k):
            c.wait()

    start_fill(0, 0)
    wait_fill(0, 0)
    scatter_copy(0, 0).start()
    start_fill(1, 1)

    NPAIR = (NFULL - 2) // 2

    def pair(p, _):
        for k in (1, 0):
            g = 2 * p + (1 if k == 1 else 2)
            wait_fill(g, k)
            scatter_copy(g, k).start()
            other = 1 - k
            scatter_copy(g - 1, other).wait()
            start_fill(g + 1, other)
        return 0

    lax.fori_loop(0, NPAIR, pair, 0)

    g_last = NFULL - 1
    wait_fill(g_last, 1)
    scatter_copy(g_last, 1).start()
    scatter_copy(g_last - 1, 0).wait()
    scatter_copy(g_last, 1).wait()

    tb = base + NFULL * CHUNK
    pltpu.sync_copy(prev_hbm.at[pl.ds(tb, TAIL)], tail_comb.at[:, pl.ds(0, D)])
    pltpu.async_copy(
        msg_hbm.at[idx_v.at[pl.ds(NFULL * CHUNK, TAIL)]],
        tail_comb.at[:, pl.ds(D, D)], gs0).wait()
    pltpu.sync_copy(tail_comb, out_hbm.at[pl.ds(tb, TAIL)])


def kernel(msg, msg_prev, edge_idx):
    idx = edge_idx[1].astype(jnp.int32)
    mesh = plsc.VectorSubcoreMesh(
        core_axis_name="c", subcore_axis_name="s",
        num_cores=NC, num_subcores=NS)
    f = pl.kernel(
        _sc_body,
        out_type=jax.ShapeDtypeStruct((N_EDGES, 2 * D), jnp.float32),
        mesh=mesh,
        scratch_types=[
            pltpu.VMEM((EPW,), jnp.int32),
            pltpu.VMEM((CHUNK, 2 * D), jnp.float32),
            pltpu.VMEM((CHUNK, 2 * D), jnp.float32),
            pltpu.VMEM((TAIL, 2 * D), jnp.float32),
            pltpu.SemaphoreType.DMA,
            pltpu.SemaphoreType.DMA,
            pltpu.SemaphoreType.DMA,
            pltpu.SemaphoreType.DMA,
            pltpu.SemaphoreType.DMA,
            pltpu.SemaphoreType.DMA,
        ],
    )
    return f(msg, msg_prev, idx)


# speedup vs baseline: 4.1120x; 1.0065x over previous
import jax
import jax.numpy as jnp
from jax import lax
from jax.experimental import pallas as pl
from jax.experimental.pallas import tpu as pltpu
from jax.experimental.pallas import tpu_sc as plsc

N_NODES = 10000
N_EDGES = 320000
D = 128
NC = 2
NS = 16
NW = NC * NS
EPW = N_EDGES // NW
CHUNK = 96
NFULL = EPW // CHUNK
TAIL = EPW - NFULL * CHUNK
NBUF = 4


def _sc_body(msg_hbm, prev_hbm, idx_hbm, out_hbm,
             idx_v, comb0, comb1, comb2, comb3, tail_comb,
             ps0, gs0, ws0, ps1, gs1, ws1,
             ps2, gs2, ws2, ps3, gs3, ws3):
    wid = lax.axis_index("s") * NC + lax.axis_index("c")
    base = wid * EPW
    pltpu.sync_copy(idx_hbm.at[pl.ds(base, EPW)], idx_v)

    bufs = (comb0, comb1, comb2, comb3)
    sems = ((ps0, gs0, ws0), (ps1, gs1, ws1), (ps2, gs2, ws2), (ps3, gs3, ws3))

    def fill_copies(g, k):
        cb = base + g * CHUNK
        buf = bufs[k]
        ps, gs, _ = sems[k]
        return (
            pltpu.make_async_copy(
                prev_hbm.at[pl.ds(cb, CHUNK)], buf.at[:, pl.ds(0, D)], ps),
            pltpu.make_async_copy(
                msg_hbm.at[idx_v.at[pl.ds(g * CHUNK, CHUNK)]],
                buf.at[:, pl.ds(D, D)], gs),
        )

    def scatter_copy(g, k):
        cb = base + g * CHUNK
        return pltpu.make_async_copy(
            bufs[k], out_hbm.at[pl.ds(cb, CHUNK)], sems[k][2])

    def start_fill(g, k):
        for c in fill_copies(g, k):
            c.start()

    def wait_fill(g, k):
        for c in fill_copies(g, k):
            c.wait()

    start_fill(0, 0)
    start_fill(1, 1)
    wait_fill(0, 0)
    scatter_copy(0, 0).start()
    start_fill(2, 2)
    wait_fill(1, 1)
    scatter_copy(1, 1).start()
    start_fill(3, 3)

    NGRP = (NFULL - 4) // 4

    def group(p, _):
        g0 = 4 * p + 2
        for j in range(4):
            g = g0 + j
            k = (2 + j) % 4
            wait_fill(g, k)
            scatter_copy(g, k).start()
            scatter_copy(g - 2, (k + 2) % 4).wait()
            start_fill(g + 2, (k + 2) % 4)
        return 0

    lax.fori_loop(0, NGRP, group, 0)

    g = NFULL - 2
    wait_fill(g, 2)
    scatter_copy(g, 2).start()
    scatter_copy(g - 2, 0).wait()
    g = NFULL - 1
    wait_fill(g, 3)
    scatter_copy(g, 3).start()
    scatter_copy(g - 2, 1).wait()
    scatter_copy(NFULL - 2, 2).wait()
    scatter_copy(NFULL - 1, 3).wait()

    tb = base + NFULL * CHUNK
    pltpu.sync_copy(prev_hbm.at[pl.ds(tb, TAIL)], tail_comb.at[:, pl.ds(0, D)])
    pltpu.async_copy(
        msg_hbm.at[idx_v.at[pl.ds(NFULL * CHUNK, TAIL)]],
        tail_comb.at[:, pl.ds(D, D)], gs0).wait()
    pltpu.sync_copy(tail_comb, out_hbm.at[pl.ds(tb, TAIL)])


def kernel(msg, msg_prev, edge_idx):
    idx = edge_idx[1].astype(jnp.int32)
    mesh = plsc.VectorSubcoreMesh(
        core_axis_name="c", subcore_axis_name="s",
        num_cores=NC, num_subcores=NS)
    f = pl.kernel(
        _sc_body,
        out_type=jax.ShapeDtypeStruct((N_EDGES, 2 * D), jnp.float32),
        mesh=mesh,
        scratch_types=[
            pltpu.VMEM((EPW,), jnp.int32),
            pltpu.VMEM((CHUNK, 2 * D), jnp.float32),
            pltpu.VMEM((CHUNK, 2 * D), jnp.float32),
            pltpu.VMEM((CHUNK, 2 * D), jnp.float32),
            pltpu.VMEM((CHUNK, 2 * D), jnp.float32),
            pltpu.VMEM((TAIL, 2 * D), jnp.float32),
        ] + [pltpu.SemaphoreType.DMA] * 12,
    )
    return f(msg, msg_prev, idx)
